# Initial kernel scaffold; baseline (speedup 1.0000x reference)
import jax
import jax.numpy as jnp
from jax.experimental import pallas as pl


def kernel(x, pos_table):
    raise NotImplementedError("write your pallas kernel here")


# speedup vs baseline: 3.4171x; 3.4171x over previous
import functools

import jax
import jax.numpy as jnp
from jax import lax
from jax.experimental import pallas as pl
from jax.experimental.pallas import tpu as pltpu
from jax.experimental.pallas import tpu_sc as plsc

_NC = 2
_NS = 16
_CH = 32


def kernel(x, pos_table):
    batch, seq_len = x.shape[0], x.shape[1]
    d_model = pos_table.shape[1]
    nw = _NC * _NS
    rows_per_w = seq_len // nw
    n_chunks = rows_per_w // _CH

    mesh = plsc.VectorSubcoreMesh(
        core_axis_name="c",
        subcore_axis_name="s",
        num_cores=_NC,
        num_subcores=_NS,
    )

    @functools.partial(
        pl.kernel,
        out_type=jax.ShapeDtypeStruct((batch, seq_len, d_model), jnp.float32),
        mesh=mesh,
        scratch_types=[
            pltpu.VMEM((_CH, d_model), jnp.float32),
            pltpu.SemaphoreType.DMA,
        ],
    )
    def broadcast_rows(table_hbm, out_hbm, buf, sem):
        wid = lax.axis_index("s") * _NC + lax.axis_index("c")
        base = wid * rows_per_w

        def body(i, carry):
            r0 = base + i * _CH
            pltpu.async_copy(table_hbm.at[pl.ds(r0, _CH)], buf, sem).wait()
            for b in range(batch):
                pltpu.sync_copy(buf, out_hbm.at[b, pl.ds(r0, _CH)])
            return carry

        lax.fori_loop(0, n_chunks, body, 0)

    return broadcast_rows(pos_table)


# speedup vs baseline: 3.6351x; 1.0638x over previous
import functools

import jax
import jax.numpy as jnp
from jax import lax
from jax.experimental import pallas as pl
from jax.experimental.pallas import tpu as pltpu
from jax.experimental.pallas import tpu_sc as plsc

_NC = 2
_NS = 16
_CH = 64


def kernel(x, pos_table):
    batch, seq_len = x.shape[0], x.shape[1]
    d_model = pos_table.shape[1]
    nw = _NC * _NS
    rows_per_w = seq_len // nw
    n_chunks = rows_per_w // _CH

    mesh = plsc.VectorSubcoreMesh(
        core_axis_name="c",
        subcore_axis_name="s",
        num_cores=_NC,
        num_subcores=_NS,
    )

    @functools.partial(
        pl.kernel,
        out_type=jax.ShapeDtypeStruct((batch, seq_len, d_model), jnp.float32),
        mesh=mesh,
        scratch_types=[
            pltpu.VMEM((_CH, d_model), jnp.float32),
            pltpu.SemaphoreType.DMA,
        ],
    )
    def broadcast_rows(table_hbm, out_hbm, buf, sem):
        wid = lax.axis_index("s") * _NC + lax.axis_index("c")
        base = wid * rows_per_w

        def body(i, carry):
            r0 = base + i * _CH
            pltpu.async_copy(table_hbm.at[pl.ds(r0, _CH)], buf, sem).wait()
            for b in range(batch):
                pltpu.sync_copy(buf, out_hbm.at[b, pl.ds(r0, _CH)])
            return carry

        lax.fori_loop(0, n_chunks, body, 0)

    return broadcast_rows(pos_table)
